# baseline (speedup 1.0000x reference)
import functools

import jax
import jax.numpy as jnp
from jax import lax
from jax.experimental import pallas as pl
from jax.experimental.pallas import tpu as pltpu
from jax.experimental.pallas import tpu_sc as plsc

N = 10000
E = 320000
NC = 2
NS = 16
EPT = E // (NC * NS)
C = 80
NCHUNK = EPT // C
ZR = 208

@functools.lru_cache(maxsize=None)
def _mesh():
    return plsc.VectorSubcoreMesh(
        core_axis_name="c", subcore_axis_name="s", num_cores=NC, num_subcores=NS
    )


def _zero_2d(ref, rows, width):
    lanes = width // 16

    def body(r, carry):
        for j in range(lanes):
            ref[r, pl.ds(j * 16, 16)] = jnp.zeros((16,), jnp.float32)
        return carry

    lax.fori_loop(0, rows, body, 0)


@functools.lru_cache(maxsize=None)
def _make_spmv(D):
    @functools.partial(
        pl.kernel,
        mesh=_mesh(),
        out_type=jax.ShapeDtypeStruct((NC, N, D), jnp.float32),
        scratch_types=[
            pltpu.VMEM((EPT,), jnp.int32),
            pltpu.VMEM((EPT,), jnp.int32),
            pltpu.VMEM((C,), jnp.int32),
            pltpu.VMEM((C,), jnp.int32),
            pltpu.VMEM((C,), jnp.int32),
            pltpu.VMEM((C,), jnp.int32),
            pltpu.VMEM((C, D), jnp.float32),
            pltpu.VMEM((C, D), jnp.float32),
            pltpu.VMEM_SHARED((N, D), jnp.float32),
            pltpu.SemaphoreType.DMA,
            pltpu.SemaphoreType.DMA,
            pltpu.SemaphoreType.DMA,
            pltpu.SemaphoreType.DMA,
        ],
    )
    def spmv(
        hs_hbm, row_hbm, col_hbm, out_hbm,
        row_all, col_all, rb0, rb1, cb0, cb1, b0, b1, acc, sem0, sem1, ssem0, ssem1,
    ):
        c = lax.axis_index("c")
        s = lax.axis_index("s")
        base = c * (E // NC) + s * EPT

        rd = pltpu.async_copy(row_hbm.at[pl.ds(base, EPT)], row_all, sem0)
        cd = pltpu.async_copy(col_hbm.at[pl.ds(base, EPT)], col_all, sem1)
        _zero_2d(b0, C, D)
        for k in range(7):
            pltpu.sync_copy(b0, acc.at[pl.ds(s * 624 + k * C, C)])
        pltpu.sync_copy(b0.at[pl.ds(0, 64)], acc.at[pl.ds(s * 624 + 7 * C, 64)])

        @pl.when(s == 0)
        def _():
            pltpu.sync_copy(b0.at[pl.ds(0, 16)], acc.at[pl.ds(16 * 624, 16)])

        rd.wait()
        cd.wait()
        plsc.subcore_barrier()

        def fill(dst, src, kk):
            for j in range(C // 16):
                dst[pl.ds(j * 16, 16)] = src[pl.ds(kk * C + j * 16, 16)]

        def start_gather(rb, buf, sem, kk):
            fill(rb, row_all, kk)
            pltpu.async_copy(hs_hbm.at[rb], buf, sem)

        def start_scatter(cb, buf, sem, kk):
            fill(cb, col_all, kk)
            pltpu.async_copy(buf, acc.at[cb], sem, add=True)

        start_gather(rb0, b0, sem0, 0)
        start_gather(rb1, b1, sem1, 1)

        def body(k2, carry):
            kk = 2 * k2
            pltpu.make_async_copy(hs_hbm.at[rb0], b0, sem0).wait()
            start_scatter(cb0, b0, ssem0, kk)
            pltpu.make_async_copy(hs_hbm.at[rb1], b1, sem1).wait()
            start_scatter(cb1, b1, ssem1, kk + 1)

            pltpu.make_async_copy(b0, acc.at[cb0], ssem0).wait()
            start_gather(rb0, b0, sem0, kk + 2)

            @pl.when(k2 < NCHUNK // 2 - 1)
            def _():
                pltpu.make_async_copy(b1, acc.at[cb1], ssem1).wait()
                start_gather(rb1, b1, sem1, kk + 3)

            return carry

        lax.fori_loop(0, NCHUNK // 2, body, 0)
        pltpu.make_async_copy(hs_hbm.at[rb0], b0, sem0).wait()
        start_scatter(cb0, b0, ssem0, NCHUNK - 1)
        pltpu.make_async_copy(b0, acc.at[cb0], ssem0).wait()
        pltpu.make_async_copy(b1, acc.at[cb1], ssem1).wait()
        plsc.subcore_barrier()

        r0 = s * 624
        pltpu.sync_copy(acc.at[pl.ds(r0, C)], b0)
        for k in range(7):
            d_out = pltpu.async_copy(
                [b0, b1][k % 2], out_hbm.at[c, pl.ds(r0 + k * C, C)], sem0
            )
            nxt = [b1, b0][k % 2]
            nn = C if k < 6 else 64
            pltpu.sync_copy(acc.at[pl.ds(r0 + (k + 1) * C, nn)], nxt.at[pl.ds(0, nn)])
            d_out.wait()
        pltpu.sync_copy(b1.at[pl.ds(0, 64)], out_hbm.at[c, pl.ds(r0 + 7 * C, 64)])

        @pl.when(s == 0)
        def _():
            pltpu.sync_copy(acc.at[pl.ds(16 * 624, 16)], b0.at[pl.ds(0, 16)])
            pltpu.sync_copy(b0.at[pl.ds(0, 16)], out_hbm.at[c, pl.ds(16 * 624, 16)])

    return spmv


@functools.lru_cache(maxsize=None)
def _make_deg():
    @functools.partial(
        pl.kernel,
        mesh=_mesh(),
        out_type=jax.ShapeDtypeStruct((NC * N,), jnp.float32),
        scratch_types=[
            pltpu.VMEM((EPT,), jnp.int32),
            pltpu.VMEM((C,), jnp.int32),
            pltpu.VMEM((C,), jnp.int32),
            pltpu.VMEM((C,), jnp.float32),
            pltpu.VMEM((624,), jnp.float32),
            pltpu.VMEM_SHARED((N,), jnp.float32),
            pltpu.SemaphoreType.DMA,
            pltpu.SemaphoreType.DMA,
        ],
    )
    def _deg_kernel(col_hbm, out_hbm, col_all, cb0, cb1, ones_v, zb, acc, sem0, sem1):
        c = lax.axis_index("c")
        s = lax.axis_index("s")
        base = c * (E // NC) + s * EPT

        cd = pltpu.async_copy(col_hbm.at[pl.ds(base, EPT)], col_all, sem0)
        for j in range(624 // 16):
            zb[pl.ds(j * 16, 16)] = jnp.zeros((16,), jnp.float32)
        for j in range(C // 16):
            ones_v[pl.ds(j * 16, 16)] = jnp.ones((16,), jnp.float32)

        pltpu.sync_copy(zb, acc.at[pl.ds(s * 624, 624)])

        @pl.when(s == 0)
        def _():
            pltpu.sync_copy(zb.at[pl.ds(0, 16)], acc.at[pl.ds(16 * 624, 16)])

        cd.wait()
        plsc.subcore_barrier()

        def fill(dst, kk):
            for j in range(C // 16):
                dst[pl.ds(j * 16, 16)] = col_all[pl.ds(kk * C + j * 16, 16)]

        def start_scatter(cb, sem, kk):
            fill(cb, kk)
            pltpu.async_copy(ones_v, acc.at[cb], sem, add=True)

        start_scatter(cb0, sem0, 0)
        start_scatter(cb1, sem1, 1)

        def body(k2, carry):
            kk = 2 * k2
            pltpu.make_async_copy(ones_v, acc.at[cb0], sem0).wait()
            start_scatter(cb0, sem0, kk + 2)
            pltpu.make_async_copy(ones_v, acc.at[cb1], sem1).wait()

            @pl.when(k2 < NCHUNK // 2 - 1)
            def _():
                start_scatter(cb1, sem1, kk + 3)

            return carry

        lax.fori_loop(0, NCHUNK // 2, body, 0)
        pltpu.make_async_copy(ones_v, acc.at[cb0], sem0).wait()
        plsc.subcore_barrier()

        pltpu.sync_copy(acc.at[pl.ds(s * 624, 624)], zb)
        pltpu.sync_copy(zb, out_hbm.at[pl.ds(c * N + s * 624, 624)])

        @pl.when(s == 0)
        def _():
            pltpu.sync_copy(acc.at[pl.ds(16 * 624, 16)], zb.at[pl.ds(0, 16)])
            pltpu.sync_copy(
                zb.at[pl.ds(0, 16)], out_hbm.at[pl.ds(c * N + 16 * 624, 16)]
            )

    return _deg_kernel


_R = 2000


def _tc_first(x, W, dv):
    def body(x_ref, w_ref, dv_ref, o_ref):
        o_ref[...] = (
            jnp.dot(x_ref[...], w_ref[...], preferred_element_type=jnp.float32)
            * dv_ref[...]
        )

    return pl.pallas_call(
        body,
        grid=(N // _R,),
        in_specs=[
            pl.BlockSpec((_R, 128), lambda i: (i, 0)),
            pl.BlockSpec((128, 128), lambda i: (0, 0)),
            pl.BlockSpec((_R, 1), lambda i: (i, 0)),
        ],
        out_specs=pl.BlockSpec((_R, 128), lambda i: (i, 0)),
        out_shape=jax.ShapeDtypeStruct((N, 128), jnp.float32),
    )(x, W, dv)


def _tc_mid(y0, y1, hs, dv, b, W):
    D2 = W.shape[1]

    def body(y0_ref, y1_ref, hs_ref, dv_ref, b_ref, w_ref, o_ref):
        z = (y0_ref[...] + y1_ref[...] + hs_ref[...]) * dv_ref[...] + b_ref[...]
        xact = jnp.maximum(z, 0.0)
        o_ref[...] = (
            jnp.dot(xact, w_ref[...], preferred_element_type=jnp.float32)
            * dv_ref[...]
        )

    return pl.pallas_call(
        body,
        grid=(N // _R,),
        in_specs=[
            pl.BlockSpec((_R, 128), lambda i: (i, 0)),
            pl.BlockSpec((_R, 128), lambda i: (i, 0)),
            pl.BlockSpec((_R, 128), lambda i: (i, 0)),
            pl.BlockSpec((_R, 1), lambda i: (i, 0)),
            pl.BlockSpec((1, 128), lambda i: (0, 0)),
            pl.BlockSpec((128, D2), lambda i: (0, 0)),
        ],
        out_specs=pl.BlockSpec((_R, D2), lambda i: (i, 0)),
        out_shape=jax.ShapeDtypeStruct((N, D2), jnp.float32),
    )(y0, y1, hs, dv, b, W)


def _tc_final(y0, y1, hs, dv, b, d_out):
    def body(y0_ref, y1_ref, hs_ref, dv_ref, b_ref, o_ref):
        z = (y0_ref[...] + y1_ref[...] + hs_ref[...]) * dv_ref[...] + b_ref[...]
        o_ref[...] = z[:, :d_out]

    return pl.pallas_call(
        body,
        grid=(N // _R,),
        in_specs=[
            pl.BlockSpec((_R, 128), lambda i: (i, 0)),
            pl.BlockSpec((_R, 128), lambda i: (i, 0)),
            pl.BlockSpec((_R, 128), lambda i: (i, 0)),
            pl.BlockSpec((_R, 1), lambda i: (i, 0)),
            pl.BlockSpec((1, 128), lambda i: (0, 0)),
        ],
        out_specs=pl.BlockSpec((_R, d_out), lambda i: (i, 0)),
        out_shape=jax.ShapeDtypeStruct((N, d_out), jnp.float32),
    )(y0, y1, hs, dv, b)


def kernel(x, edge_index, W1, b1, W2, b2, W3, b3):
    row = edge_index[0]
    col = edge_index[1]

    d2 = _make_deg()(col).reshape(NC, N)
    dinv = lax.rsqrt(1.0 + d2[0] + d2[1])
    dv = dinv[:, None]

    hs1 = _tc_first(x, W1, dv)
    y1 = _make_spmv(128)(hs1, row, col)
    hs2 = _tc_mid(y1[0], y1[1], hs1, dv, b1.reshape(1, -1), W2)
    y2 = _make_spmv(128)(hs2, row, col)
    W3p = jnp.pad(W3, ((0, 0), (0, 128 - W3.shape[1])))
    b3p = jnp.pad(b3, (0, 128 - b3.shape[0])).reshape(1, -1)
    hs3 = _tc_mid(y2[0], y2[1], hs2, dv, b2.reshape(1, -1), W3p)
    y3 = _make_spmv(128)(hs3, row, col)
    mu = _tc_final(y3[0], y3[1], hs3, dv, b3p, W3.shape[1])
    return (mu, mu)


# speedup vs baseline: 1.4052x; 1.4052x over previous
import functools

import jax
import jax.numpy as jnp
from jax import lax
from jax.experimental import pallas as pl
from jax.experimental.pallas import tpu as pltpu
from jax.experimental.pallas import tpu_sc as plsc

N = 10000
E = 320000
NC = 2
NS = 16
EPT = E // (NC * NS)
C = 80
NCHUNK = EPT // C
ZR = 208

@functools.lru_cache(maxsize=None)
def _mesh():
    return plsc.VectorSubcoreMesh(
        core_axis_name="c", subcore_axis_name="s", num_cores=NC, num_subcores=NS
    )


def _zero_2d(ref, rows, width):
    lanes = width // 16

    def body(r, carry):
        for j in range(lanes):
            ref[r, pl.ds(j * 16, 16)] = jnp.zeros((16,), jnp.float32)
        return carry

    lax.fori_loop(0, rows, body, 0)


CS = 64
NCHS = EPT // CS
TAIL = EPT - NCHS * CS

_SEGS = [(k * CS, CS) for k in range(624 // CS)] + [(624 - 624 % CS, 624 % CS)]


@functools.lru_cache(maxsize=None)
def _make_spmv(D):
    @functools.partial(
        pl.kernel,
        mesh=_mesh(),
        out_type=jax.ShapeDtypeStruct((NC, N, D), jnp.float32),
        scratch_types=[
            pltpu.VMEM((EPT,), jnp.int32),
            pltpu.VMEM((EPT,), jnp.int32),
            pltpu.VMEM((CS,), jnp.int32),
            pltpu.VMEM((CS,), jnp.int32),
            pltpu.VMEM((CS,), jnp.int32),
            pltpu.VMEM((16,), jnp.int32),
            pltpu.VMEM((CS, D), jnp.float32),
            pltpu.VMEM((CS, D), jnp.float32),
            pltpu.VMEM((CS, D), jnp.float32),
            pltpu.VMEM_SHARED((N, D), jnp.float32),
            pltpu.SemaphoreType.DMA,
            pltpu.SemaphoreType.DMA,
            pltpu.SemaphoreType.DMA,
        ],
    )
    def spmv(
        hs_hbm, row_hbm, col_hbm, out_hbm,
        row_all, col_all, cb0, cb1, cb2, cbt, b0, b1, b2, acc, sem0, sem1, sem2,
    ):
        c = lax.axis_index("c")
        s = lax.axis_index("s")
        base = c * (E // NC) + s * EPT

        rd = pltpu.async_copy(row_hbm.at[pl.ds(base, EPT)], row_all, sem0)
        cd = pltpu.async_copy(col_hbm.at[pl.ds(base, EPT)], col_all, sem1)
        _zero_2d(b0, CS, D)
        for off, nn in _SEGS:
            pltpu.sync_copy(b0.at[pl.ds(0, nn)], acc.at[pl.ds(s * 624 + off, nn)])

        @pl.when(s == 0)
        def _():
            pltpu.sync_copy(b0.at[pl.ds(0, 16)], acc.at[pl.ds(16 * 624, 16)])

        rd.wait()
        cd.wait()
        plsc.subcore_barrier()

        def fill(dst, src, off, nn):
            for j in range(nn // 16):
                dst[pl.ds(j * 16, 16)] = src[pl.ds(off + j * 16, 16)]

        def start_gather(buf, sem, kk):
            pltpu.async_copy(hs_hbm.at[row_all.at[pl.ds(kk * CS, CS)]], buf, sem)

        def scatter(cb, buf, kk):
            fill(cb, col_all, kk * CS, CS)
            pltpu.sync_copy(buf, acc.at[cb], add=True)

        slots = ((cb0, b0, sem0), (cb1, b1, sem1), (cb2, b2, sem2))
        for sl in range(3):
            start_gather(slots[sl][1], slots[sl][2], sl)

        def body(k3, carry):
            for sl in range(3):
                kk = 3 * k3 + sl
                cb, buf, sem = slots[sl]
                pltpu.make_async_copy(hs_hbm.at[cb], buf, sem).wait()
                scatter(cb, buf, kk)

                @pl.when(kk + 3 < NCHS)
                def _():
                    start_gather(buf, sem, kk + 3)

            return carry

        lax.fori_loop(0, NCHS // 3, body, 0)
        pltpu.async_copy(
            hs_hbm.at[row_all.at[pl.ds(NCHS * CS, TAIL)]],
            b0.at[pl.ds(0, TAIL)], sem0,
        )
        fill(cbt, col_all, NCHS * CS, TAIL)
        pltpu.make_async_copy(hs_hbm.at[cbt], b0.at[pl.ds(0, TAIL)], sem0).wait()
        pltpu.sync_copy(b0.at[pl.ds(0, TAIL)], acc.at[cbt], add=True)
        plsc.subcore_barrier()

        r0 = s * 624
        pltpu.sync_copy(acc.at[pl.ds(r0, CS)], b0)
        for k, (off, nn) in enumerate(_SEGS):
            cur, nxt = ((b0, b1), (b1, b0))[k % 2]
            d_out = pltpu.async_copy(
                cur.at[pl.ds(0, nn)], out_hbm.at[c, pl.ds(r0 + off, nn)], sem0
            )
            if k + 1 < len(_SEGS):
                off2, nn2 = _SEGS[k + 1]
                pltpu.sync_copy(
                    acc.at[pl.ds(r0 + off2, nn2)], nxt.at[pl.ds(0, nn2)]
                )
            d_out.wait()

        @pl.when(s == 0)
        def _():
            pltpu.sync_copy(acc.at[pl.ds(16 * 624, 16)], b0.at[pl.ds(0, 16)])
            pltpu.sync_copy(b0.at[pl.ds(0, 16)], out_hbm.at[c, pl.ds(16 * 624, 16)])

    return spmv


@functools.lru_cache(maxsize=None)
def _make_deg():
    @functools.partial(
        pl.kernel,
        mesh=_mesh(),
        out_type=jax.ShapeDtypeStruct((NC * N,), jnp.float32),
        scratch_types=[
            pltpu.VMEM((EPT,), jnp.int32),
            pltpu.VMEM((C,), jnp.int32),
            pltpu.VMEM((C,), jnp.int32),
            pltpu.VMEM((C,), jnp.float32),
            pltpu.VMEM((624,), jnp.float32),
            pltpu.VMEM_SHARED((N,), jnp.float32),
            pltpu.SemaphoreType.DMA,
            pltpu.SemaphoreType.DMA,
        ],
    )
    def _deg_kernel(col_hbm, out_hbm, col_all, cb0, cb1, ones_v, zb, acc, sem0, sem1):
        c = lax.axis_index("c")
        s = lax.axis_index("s")
        base = c * (E // NC) + s * EPT

        cd = pltpu.async_copy(col_hbm.at[pl.ds(base, EPT)], col_all, sem0)
        for j in range(624 // 16):
            zb[pl.ds(j * 16, 16)] = jnp.zeros((16,), jnp.float32)
        for j in range(C // 16):
            ones_v[pl.ds(j * 16, 16)] = jnp.ones((16,), jnp.float32)

        pltpu.sync_copy(zb, acc.at[pl.ds(s * 624, 624)])

        @pl.when(s == 0)
        def _():
            pltpu.sync_copy(zb.at[pl.ds(0, 16)], acc.at[pl.ds(16 * 624, 16)])

        cd.wait()
        plsc.subcore_barrier()

        def fill(dst, kk):
            for j in range(C // 16):
                dst[pl.ds(j * 16, 16)] = col_all[pl.ds(kk * C + j * 16, 16)]

        def start_scatter(cb, sem, kk):
            fill(cb, kk)
            pltpu.async_copy(ones_v, acc.at[cb], sem, add=True)

        start_scatter(cb0, sem0, 0)
        start_scatter(cb1, sem1, 1)

        def body(k2, carry):
            kk = 2 * k2
            pltpu.make_async_copy(ones_v, acc.at[cb0], sem0).wait()
            start_scatter(cb0, sem0, kk + 2)
            pltpu.make_async_copy(ones_v, acc.at[cb1], sem1).wait()

            @pl.when(k2 < NCHUNK // 2 - 1)
            def _():
                start_scatter(cb1, sem1, kk + 3)

            return carry

        lax.fori_loop(0, NCHUNK // 2, body, 0)
        pltpu.make_async_copy(ones_v, acc.at[cb0], sem0).wait()
        plsc.subcore_barrier()

        pltpu.sync_copy(acc.at[pl.ds(s * 624, 624)], zb)
        pltpu.sync_copy(zb, out_hbm.at[pl.ds(c * N + s * 624, 624)])

        @pl.when(s == 0)
        def _():
            pltpu.sync_copy(acc.at[pl.ds(16 * 624, 16)], zb.at[pl.ds(0, 16)])
            pltpu.sync_copy(
                zb.at[pl.ds(0, 16)], out_hbm.at[pl.ds(c * N + 16 * 624, 16)]
            )

    return _deg_kernel


_R = 2000


def _tc_first(x, W, dv):
    def body(x_ref, w_ref, dv_ref, o_ref):
        o_ref[...] = (
            jnp.dot(x_ref[...], w_ref[...], preferred_element_type=jnp.float32)
            * dv_ref[...]
        )

    return pl.pallas_call(
        body,
        grid=(N // _R,),
        in_specs=[
            pl.BlockSpec((_R, 128), lambda i: (i, 0)),
            pl.BlockSpec((128, 128), lambda i: (0, 0)),
            pl.BlockSpec((_R, 1), lambda i: (i, 0)),
        ],
        out_specs=pl.BlockSpec((_R, 128), lambda i: (i, 0)),
        out_shape=jax.ShapeDtypeStruct((N, 128), jnp.float32),
    )(x, W, dv)


def _tc_mid(y0, y1, hs, dv, b, W):
    D2 = W.shape[1]

    def body(y0_ref, y1_ref, hs_ref, dv_ref, b_ref, w_ref, o_ref):
        z = (y0_ref[...] + y1_ref[...] + hs_ref[...]) * dv_ref[...] + b_ref[...]
        xact = jnp.maximum(z, 0.0)
        o_ref[...] = (
            jnp.dot(xact, w_ref[...], preferred_element_type=jnp.float32)
            * dv_ref[...]
        )

    return pl.pallas_call(
        body,
        grid=(N // _R,),
        in_specs=[
            pl.BlockSpec((_R, 128), lambda i: (i, 0)),
            pl.BlockSpec((_R, 128), lambda i: (i, 0)),
            pl.BlockSpec((_R, 128), lambda i: (i, 0)),
            pl.BlockSpec((_R, 1), lambda i: (i, 0)),
            pl.BlockSpec((1, 128), lambda i: (0, 0)),
            pl.BlockSpec((128, D2), lambda i: (0, 0)),
        ],
        out_specs=pl.BlockSpec((_R, D2), lambda i: (i, 0)),
        out_shape=jax.ShapeDtypeStruct((N, D2), jnp.float32),
    )(y0, y1, hs, dv, b, W)


def _tc_final(y0, y1, hs, dv, b, d_out):
    def body(y0_ref, y1_ref, hs_ref, dv_ref, b_ref, o_ref):
        z = (y0_ref[...] + y1_ref[...] + hs_ref[...]) * dv_ref[...] + b_ref[...]
        o_ref[...] = z[:, :d_out]

    return pl.pallas_call(
        body,
        grid=(N // _R,),
        in_specs=[
            pl.BlockSpec((_R, 128), lambda i: (i, 0)),
            pl.BlockSpec((_R, 128), lambda i: (i, 0)),
            pl.BlockSpec((_R, 128), lambda i: (i, 0)),
            pl.BlockSpec((_R, 1), lambda i: (i, 0)),
            pl.BlockSpec((1, 128), lambda i: (0, 0)),
        ],
        out_specs=pl.BlockSpec((_R, d_out), lambda i: (i, 0)),
        out_shape=jax.ShapeDtypeStruct((N, d_out), jnp.float32),
    )(y0, y1, hs, dv, b)


def kernel(x, edge_index, W1, b1, W2, b2, W3, b3):
    row = edge_index[0]
    col = edge_index[1]

    d2 = _make_deg()(col).reshape(NC, N)
    dinv = lax.rsqrt(1.0 + d2[0] + d2[1])
    dv = dinv[:, None]

    hs1 = _tc_first(x, W1, dv)
    y1 = _make_spmv(128)(hs1, row, col)
    hs2 = _tc_mid(y1[0], y1[1], hs1, dv, b1.reshape(1, -1), W2)
    y2 = _make_spmv(128)(hs2, row, col)
    W3p = jnp.pad(W3, ((0, 0), (0, 128 - W3.shape[1])))
    b3p = jnp.pad(b3, (0, 128 - b3.shape[0])).reshape(1, -1)
    hs3 = _tc_mid(y2[0], y2[1], hs2, dv, b2.reshape(1, -1), W3p)
    y3 = _make_spmv(128)(hs3, row, col)
    mu = _tc_final(y3[0], y3[1], hs3, dv, b3p, W3.shape[1])
    return (mu, mu)


# speedup vs baseline: 1.4844x; 1.0563x over previous
import functools

import jax
import jax.numpy as jnp
from jax import lax
from jax.experimental import pallas as pl
from jax.experimental.pallas import tpu as pltpu
from jax.experimental.pallas import tpu_sc as plsc

N = 10000
E = 320000
NC = 2
NS = 16
EPT = E // (NC * NS)
C = 80
NCHUNK = EPT // C
ZR = 208

@functools.lru_cache(maxsize=None)
def _mesh():
    return plsc.VectorSubcoreMesh(
        core_axis_name="c", subcore_axis_name="s", num_cores=NC, num_subcores=NS
    )


def _zero_2d(ref, rows, width):
    lanes = width // 16

    def body(r, carry):
        for j in range(lanes):
            ref[r, pl.ds(j * 16, 16)] = jnp.zeros((16,), jnp.float32)
        return carry

    lax.fori_loop(0, rows, body, 0)


CS = 64
NCHS = EPT // CS
TAIL = EPT - NCHS * CS

_SEGS = [(k * CS, CS) for k in range(624 // CS)] + [(624 - 624 % CS, 624 % CS)]


@functools.lru_cache(maxsize=None)
def _make_spmv(D):
    @functools.partial(
        pl.kernel,
        mesh=_mesh(),
        out_type=jax.ShapeDtypeStruct((NC, N, D), jnp.float32),
        scratch_types=[
            pltpu.VMEM((EPT,), jnp.int32),
            pltpu.VMEM((EPT,), jnp.int32),
            pltpu.VMEM((CS,), jnp.int32),
            pltpu.VMEM((CS,), jnp.int32),
            pltpu.VMEM((CS,), jnp.int32),
            pltpu.VMEM((16,), jnp.int32),
            pltpu.VMEM((CS, D), jnp.float32),
            pltpu.VMEM((CS, D), jnp.float32),
            pltpu.VMEM((CS, D), jnp.float32),
            pltpu.VMEM_SHARED((N, D), jnp.float32),
            pltpu.SemaphoreType.DMA,
            pltpu.SemaphoreType.DMA,
            pltpu.SemaphoreType.DMA,
        ],
    )
    def spmv(
        hs_hbm, row_hbm, col_hbm, out_hbm,
        row_all, col_all, cb0, cb1, cb2, cbt, b0, b1, b2, acc, sem0, sem1, sem2,
    ):
        c = lax.axis_index("c")
        s = lax.axis_index("s")
        base = c * (E // NC) + s * EPT

        rd = pltpu.async_copy(row_hbm.at[pl.ds(base, EPT)], row_all, sem0)
        cd = pltpu.async_copy(col_hbm.at[pl.ds(base, EPT)], col_all, sem1)
        _zero_2d(b0, CS, D)
        for off, nn in _SEGS:
            pltpu.sync_copy(b0.at[pl.ds(0, nn)], acc.at[pl.ds(s * 624 + off, nn)])

        @pl.when(s == 0)
        def _():
            pltpu.sync_copy(b0.at[pl.ds(0, 16)], acc.at[pl.ds(16 * 624, 16)])

        rd.wait()
        cd.wait()
        plsc.subcore_barrier()

        def fill(dst, src, off, nn):
            for j in range(nn // 16):
                dst[pl.ds(j * 16, 16)] = src[pl.ds(off + j * 16, 16)]

        def start_gather(buf, sem, kk):
            pltpu.async_copy(hs_hbm.at[row_all.at[pl.ds(kk * CS, CS)]], buf, sem)

        def scatter(cb, buf, kk):
            fill(cb, col_all, kk * CS, CS)
            pltpu.sync_copy(buf, acc.at[cb], add=True)

        slots = ((cb0, b0, sem0), (cb1, b1, sem1), (cb2, b2, sem2))
        for sl in range(3):
            start_gather(slots[sl][1], slots[sl][2], sl)

        def body(k3, carry):
            for sl in range(3):
                kk = 3 * k3 + sl
                cb, buf, sem = slots[sl]
                pltpu.make_async_copy(hs_hbm.at[cb], buf, sem).wait()
                scatter(cb, buf, kk)

                @pl.when(kk + 3 < NCHS)
                def _():
                    start_gather(buf, sem, kk + 3)

            return carry

        lax.fori_loop(0, NCHS // 3, body, 0)
        pltpu.async_copy(
            hs_hbm.at[row_all.at[pl.ds(NCHS * CS, TAIL)]],
            b0.at[pl.ds(0, TAIL)], sem0,
        )
        fill(cbt, col_all, NCHS * CS, TAIL)
        pltpu.make_async_copy(hs_hbm.at[cbt], b0.at[pl.ds(0, TAIL)], sem0).wait()
        pltpu.sync_copy(b0.at[pl.ds(0, TAIL)], acc.at[cbt], add=True)
        plsc.subcore_barrier()

        r0 = s * 624
        pltpu.sync_copy(acc.at[pl.ds(r0, CS)], b0)
        for k, (off, nn) in enumerate(_SEGS):
            cur, nxt = ((b0, b1), (b1, b0))[k % 2]
            d_out = pltpu.async_copy(
                cur.at[pl.ds(0, nn)], out_hbm.at[c, pl.ds(r0 + off, nn)], sem0
            )
            if k + 1 < len(_SEGS):
                off2, nn2 = _SEGS[k + 1]
                pltpu.sync_copy(
                    acc.at[pl.ds(r0 + off2, nn2)], nxt.at[pl.ds(0, nn2)]
                )
            d_out.wait()

        @pl.when(s == 0)
        def _():
            pltpu.sync_copy(acc.at[pl.ds(16 * 624, 16)], b0.at[pl.ds(0, 16)])
            pltpu.sync_copy(b0.at[pl.ds(0, 16)], out_hbm.at[c, pl.ds(16 * 624, 16)])

    return spmv


@functools.lru_cache(maxsize=None)
def _make_deg():
    @functools.partial(
        pl.kernel,
        mesh=_mesh(),
        out_type=jax.ShapeDtypeStruct((NC * N,), jnp.float32),
        scratch_types=[
            pltpu.VMEM((EPT,), jnp.int32),
            pltpu.VMEM((C,), jnp.int32),
            pltpu.VMEM((C,), jnp.int32),
            pltpu.VMEM((C,), jnp.float32),
            pltpu.VMEM((624,), jnp.float32),
            pltpu.VMEM_SHARED((N,), jnp.float32),
            pltpu.SemaphoreType.DMA,
            pltpu.SemaphoreType.DMA,
        ],
    )
    def _deg_kernel(col_hbm, out_hbm, col_all, cb0, cb1, ones_v, zb, acc, sem0, sem1):
        c = lax.axis_index("c")
        s = lax.axis_index("s")
        base = c * (E // NC) + s * EPT

        cd = pltpu.async_copy(col_hbm.at[pl.ds(base, EPT)], col_all, sem0)
        for j in range(624 // 16):
            zb[pl.ds(j * 16, 16)] = jnp.zeros((16,), jnp.float32)
        for j in range(C // 16):
            ones_v[pl.ds(j * 16, 16)] = jnp.ones((16,), jnp.float32)

        pltpu.sync_copy(zb, acc.at[pl.ds(s * 624, 624)])

        @pl.when(s == 0)
        def _():
            pltpu.sync_copy(zb.at[pl.ds(0, 16)], acc.at[pl.ds(16 * 624, 16)])

        cd.wait()
        plsc.subcore_barrier()

        def fill(dst, kk):
            for j in range(C // 16):
                dst[pl.ds(j * 16, 16)] = col_all[pl.ds(kk * C + j * 16, 16)]

        def start_scatter(cb, sem, kk):
            fill(cb, kk)
            pltpu.async_copy(ones_v, acc.at[cb], sem, add=True)

        start_scatter(cb0, sem0, 0)
        start_scatter(cb1, sem1, 1)

        def body(k2, carry):
            kk = 2 * k2
            pltpu.make_async_copy(ones_v, acc.at[cb0], sem0).wait()
            start_scatter(cb0, sem0, kk + 2)
            pltpu.make_async_copy(ones_v, acc.at[cb1], sem1).wait()

            @pl.when(k2 < NCHUNK // 2 - 1)
            def _():
                start_scatter(cb1, sem1, kk + 3)

            return carry

        lax.fori_loop(0, NCHUNK // 2, body, 0)
        pltpu.make_async_copy(ones_v, acc.at[cb0], sem0).wait()
        plsc.subcore_barrier()

        pltpu.sync_copy(acc.at[pl.ds(s * 624, 624)], zb)
        pltpu.sync_copy(zb, out_hbm.at[pl.ds(c * N + s * 624, 624)])

        @pl.when(s == 0)
        def _():
            pltpu.sync_copy(acc.at[pl.ds(16 * 624, 16)], zb.at[pl.ds(0, 16)])
            pltpu.sync_copy(
                zb.at[pl.ds(0, 16)], out_hbm.at[pl.ds(c * N + 16 * 624, 16)]
            )

    return _deg_kernel


_R = 2000


def _tc_first(x, W, dv):
    def body(x_ref, w_ref, dv_ref, o_ref):
        o_ref[...] = (
            jnp.dot(x_ref[...], w_ref[...], preferred_element_type=jnp.float32)
            * dv_ref[...]
        )

    return pl.pallas_call(
        body,
        grid=(N // _R,),
        in_specs=[
            pl.BlockSpec((_R, 128), lambda i: (i, 0)),
            pl.BlockSpec((128, 128), lambda i: (0, 0)),
            pl.BlockSpec((_R, 1), lambda i: (i, 0)),
        ],
        out_specs=pl.BlockSpec((_R, 128), lambda i: (i, 0)),
        out_shape=jax.ShapeDtypeStruct((N, 128), jnp.float32),
    )(x, W, dv)


def _tc_mid(y, hs, dv, b, W):
    D2 = W.shape[1]

    def body(y_ref, hs_ref, dv_ref, b_ref, w_ref, o_ref):
        z = (y_ref[0] + y_ref[1] + hs_ref[...]) * dv_ref[...] + b_ref[...]
        xact = jnp.maximum(z, 0.0)
        o_ref[...] = (
            jnp.dot(xact, w_ref[...], preferred_element_type=jnp.float32)
            * dv_ref[...]
        )

    return pl.pallas_call(
        body,
        grid=(N // _R,),
        in_specs=[
            pl.BlockSpec((2, _R, 128), lambda i: (0, i, 0)),
            pl.BlockSpec((_R, 128), lambda i: (i, 0)),
            pl.BlockSpec((_R, 1), lambda i: (i, 0)),
            pl.BlockSpec((1, 128), lambda i: (0, 0)),
            pl.BlockSpec((128, D2), lambda i: (0, 0)),
        ],
        out_specs=pl.BlockSpec((_R, D2), lambda i: (i, 0)),
        out_shape=jax.ShapeDtypeStruct((N, D2), jnp.float32),
    )(y, hs, dv, b, W)


def _tc_final(y, hs, dv, b, d_out):
    def body(y_ref, hs_ref, dv_ref, b_ref, o_ref):
        z = (y_ref[0] + y_ref[1] + hs_ref[...]) * dv_ref[...] + b_ref[...]
        o_ref[...] = z[:, :d_out]

    return pl.pallas_call(
        body,
        grid=(N // _R,),
        in_specs=[
            pl.BlockSpec((2, _R, 128), lambda i: (0, i, 0)),
            pl.BlockSpec((_R, 128), lambda i: (i, 0)),
            pl.BlockSpec((_R, 1), lambda i: (i, 0)),
            pl.BlockSpec((1, 128), lambda i: (0, 0)),
        ],
        out_specs=pl.BlockSpec((_R, d_out), lambda i: (i, 0)),
        out_shape=jax.ShapeDtypeStruct((N, d_out), jnp.float32),
    )(y, hs, dv, b)


def kernel(x, edge_index, W1, b1, W2, b2, W3, b3):
    row = edge_index[0]
    col = edge_index[1]

    d2 = _make_deg()(col).reshape(NC, N)
    dinv = lax.rsqrt(1.0 + d2[0] + d2[1])
    dv = dinv[:, None]

    hs1 = _tc_first(x, W1, dv)
    y1 = _make_spmv(128)(hs1, row, col)
    hs2 = _tc_mid(y1, hs1, dv, b1.reshape(1, -1), W2)
    y2 = _make_spmv(128)(hs2, row, col)
    W3p = jnp.pad(W3, ((0, 0), (0, 128 - W3.shape[1])))
    b3p = jnp.pad(b3, (0, 128 - b3.shape[0])).reshape(1, -1)
    hs3 = _tc_mid(y2, hs2, dv, b2.reshape(1, -1), W3p)
    y3 = _make_spmv(128)(hs3, row, col)
    mu = _tc_final(y3, hs3, dv, b3p, W3.shape[1])
    return (mu, mu)
